# Initial kernel scaffold; baseline (speedup 1.0000x reference)
import jax
import jax.numpy as jnp
from jax.experimental import pallas as pl


def kernel(v_x, v_edge_index, e_x, e_edge_index, enc_v, enc_e, layers, dec_v, dec_e):
    raise NotImplementedError("write your pallas kernel here")


# speedup vs baseline: 1.6400x; 1.6400x over previous
import jax
import jax.numpy as jnp
from jax import lax
from jax.experimental import pallas as pl
from jax.experimental.pallas import tpu as pltpu
from jax.experimental.pallas import tpu_sc as plsc

N = 10000
E = 320000
H = 128
HH = H // 2
EPS = 1e-7

NC = 2
NS = 16
IW = 128
TILE_ROWS = 160
E_PAD = NS * TILE_ROWS * IW
N_ACC = N + 8
ROWS_PT = 624
ROWS_TAIL = N - NS * ROWS_PT

_PREC = lax.Precision.HIGHEST


def _sc_agg_body(vs_hbm, es_hbm, src_hbm, dst_hbm, zero_hbm, out_hbm,
                 sidx, didx, g, eb, acc, sem):
    cid = lax.axis_index("c")
    sid = lax.axis_index("s")

    pltpu.sync_copy(zero_hbm.at[pl.ds(sid * ROWS_PT, ROWS_PT)],
                    acc.at[pl.ds(sid * ROWS_PT, ROWS_PT)])

    @pl.when(sid == 0)
    def _():
        tail = N_ACC - NS * ROWS_PT
        pltpu.sync_copy(zero_hbm.at[pl.ds(NS * ROWS_PT, tail)],
                        acc.at[pl.ds(NS * ROWS_PT, tail)])

    pltpu.sync_copy(src_hbm.at[pl.ds(sid * TILE_ROWS, TILE_ROWS)], sidx)
    pltpu.sync_copy(dst_hbm.at[pl.ds(sid * TILE_ROWS, TILE_ROWS)], didx)
    plsc.subcore_barrier()

    vtab = vs_hbm.at[cid]

    @pl.loop(0, TILE_ROWS)
    def _step(j):
        pltpu.async_copy(vtab.at[sidx.at[j]], g, sem).wait()
        base = (sid * TILE_ROWS + j) * IW
        pltpu.sync_copy(es_hbm.at[cid, pl.ds(base, IW)], eb)

        @pl.loop(0, IW)
        def _row(r):
            for c16 in range(HH // 16):
                slc = (pl.ds(r, 1), pl.ds(c16 * 16, 16))
                g.at[slc][...] = (
                    jnp.maximum(g.at[slc][...] + eb.at[slc][...], 0.0) + EPS)

        pltpu.sync_copy(g, acc.at[didx.at[j]], add=True)

    plsc.subcore_barrier()
    pltpu.sync_copy(acc.at[pl.ds(sid * ROWS_PT, ROWS_PT)],
                    out_hbm.at[cid, pl.ds(sid * ROWS_PT, ROWS_PT)])

    @pl.when(sid == 0)
    def _():
        pltpu.sync_copy(acc.at[pl.ds(NS * ROWS_PT, ROWS_TAIL)],
                        out_hbm.at[cid, pl.ds(NS * ROWS_PT, ROWS_TAIL)])


def _sc_agg(vs, es, src2, dst2, zero):
    mesh = plsc.VectorSubcoreMesh(core_axis_name="c", subcore_axis_name="s")
    k = pl.kernel(
        _sc_agg_body,
        mesh=mesh,
        out_type=jax.ShapeDtypeStruct((NC, N, HH), jnp.float32),
        scratch_types=[
            pltpu.VMEM((TILE_ROWS, IW), jnp.int32),
            pltpu.VMEM((TILE_ROWS, IW), jnp.int32),
            pltpu.VMEM((IW, HH), jnp.float32),
            pltpu.VMEM((IW, HH), jnp.float32),
            pltpu.VMEM_SHARED((N_ACC, HH), jnp.float32),
            pltpu.SemaphoreType.DMA,
        ],
        compiler_params=pltpu.CompilerParams(use_tc_tiling_on_sc=False),
    )
    return k(vs, es, src2, dst2, zero)


def _enc_body(x_ref, w1_ref, b1_ref, w2_ref, b2_ref, o_ref):
    h = jnp.maximum(
        jnp.dot(x_ref[...], w1_ref[...],
                preferred_element_type=jnp.float32, precision=_PREC)
        + b1_ref[...], 0.0)
    y = (jnp.dot(h, w2_ref[...],
                 preferred_element_type=jnp.float32, precision=_PREC)
         + b2_ref[...])
    o_ref[0] = y[:, :HH]
    o_ref[1] = y[:, HH:]


def _encode(x, p, bm, rows_out):
    W1, b1, W2, b2 = p
    M, din = x.shape
    dh = W1.shape[1]
    return pl.pallas_call(
        _enc_body,
        grid=(M // bm,),
        in_specs=[
            pl.BlockSpec((bm, din), lambda i: (i, 0)),
            pl.BlockSpec((din, dh), lambda i: (0, 0)),
            pl.BlockSpec((1, dh), lambda i: (0, 0)),
            pl.BlockSpec((dh, H), lambda i: (0, 0)),
            pl.BlockSpec((1, H), lambda i: (0, 0)),
        ],
        out_specs=pl.BlockSpec((2, bm, HH), lambda i: (0, i, 0)),
        out_shape=jax.ShapeDtypeStruct((2, rows_out, HH), jnp.float32),
    )(x, W1, b1.reshape(1, dh), W2, b2.reshape(1, H))


def _layer_body(a_ref, v_ref, w1_ref, b1_ref, w2_ref, b2_ref, o_ref):
    w1 = w1_ref[...]
    h = jnp.maximum(
        jnp.dot(a_ref[0], w1[:HH, :],
                preferred_element_type=jnp.float32, precision=_PREC)
        + jnp.dot(a_ref[1], w1[HH:, :],
                  preferred_element_type=jnp.float32, precision=_PREC)
        + b1_ref[...], 0.0)
    y = (jnp.dot(h, w2_ref[...],
                 preferred_element_type=jnp.float32, precision=_PREC)
         + b2_ref[...])
    o_ref[0] = v_ref[0] + y[:, :HH]
    o_ref[1] = v_ref[1] + y[:, HH:]


def _layer_mlp(aggs, vs, p, bm):
    W1, b1, W2, b2 = p
    return pl.pallas_call(
        _layer_body,
        grid=(N // bm,),
        in_specs=[
            pl.BlockSpec((2, bm, HH), lambda i: (0, i, 0)),
            pl.BlockSpec((2, bm, HH), lambda i: (0, i, 0)),
            pl.BlockSpec((H, H), lambda i: (0, 0)),
            pl.BlockSpec((1, H), lambda i: (0, 0)),
            pl.BlockSpec((H, H), lambda i: (0, 0)),
            pl.BlockSpec((1, H), lambda i: (0, 0)),
        ],
        out_specs=pl.BlockSpec((2, bm, HH), lambda i: (0, i, 0)),
        out_shape=jax.ShapeDtypeStruct((2, N, HH), jnp.float32),
    )(aggs, vs, W1, b1.reshape(1, H), W2, b2.reshape(1, H))


def _dec_body(x_ref, w1_ref, b1_ref, w2_ref, b2_ref, o_ref):
    w1 = w1_ref[...]
    h = jnp.maximum(
        jnp.dot(x_ref[0], w1[:HH, :],
                preferred_element_type=jnp.float32, precision=_PREC)
        + jnp.dot(x_ref[1], w1[HH:, :],
                  preferred_element_type=jnp.float32, precision=_PREC)
        + b1_ref[...], 0.0)
    o_ref[...] = (
        jnp.dot(h, w2_ref[...],
                preferred_element_type=jnp.float32, precision=_PREC)
        + b2_ref[...])


def _decode(xs, p, bm, rows):
    W1, b1, W2, b2 = p
    dh = W1.shape[1]
    dout = W2.shape[1]
    dpad = 8
    W2p = jnp.zeros((dh, dpad), jnp.float32).at[:, :dout].set(W2)
    b2p = jnp.zeros((1, dpad), jnp.float32).at[0, :dout].set(b2)
    y = pl.pallas_call(
        _dec_body,
        grid=(rows // bm,),
        in_specs=[
            pl.BlockSpec((2, bm, HH), lambda i: (0, i, 0)),
            pl.BlockSpec((H, dh), lambda i: (0, 0)),
            pl.BlockSpec((1, dh), lambda i: (0, 0)),
            pl.BlockSpec((dh, dpad), lambda i: (0, 0)),
            pl.BlockSpec((1, dpad), lambda i: (0, 0)),
        ],
        out_specs=pl.BlockSpec((bm, dpad), lambda i: (i, 0)),
        out_shape=jax.ShapeDtypeStruct((rows, dpad), jnp.float32),
    )(xs, W1, b1.reshape(1, dh), W2p, b2p)
    return y[:, :dout]


def kernel(v_x, v_edge_index, e_x, e_edge_index, enc_v, enc_e, layers,
           dec_v, dec_e):
    src = v_edge_index[0]
    dst = v_edge_index[1]
    src2 = jnp.zeros((E_PAD,), jnp.int32).at[:E].set(src).reshape(E_PAD // IW, IW)
    dst2 = jnp.full((E_PAD,), N, jnp.int32).at[:E].set(dst).reshape(E_PAD // IW, IW)
    zero = jnp.zeros((N_ACC, HH), jnp.float32)

    vs = _encode(v_x, enc_v, bm=1000, rows_out=N)
    es = _encode(e_x, enc_e, bm=2000, rows_out=E_PAD)

    for (pv, _pe) in layers:
        aggs = _sc_agg(vs, es, src2, dst2, zero)
        vs = _layer_mlp(aggs, vs, pv, bm=1000)

    node_out = _decode(vs, dec_v, bm=1000, rows=N)
    edge_out = _decode(es, dec_e, bm=2000, rows=E)
    return (node_out, edge_out)


# speedup vs baseline: 2.1181x; 1.2915x over previous
import jax
import jax.numpy as jnp
from jax import lax
from jax.experimental import pallas as pl
from jax.experimental.pallas import tpu as pltpu
from jax.experimental.pallas import tpu_sc as plsc

N = 10000
E = 320000
H = 128
HH = H // 2
EPS = 1e-7

NC = 2
NS = 16
IW = 128
TILE_ROWS = 160
E_PAD = NS * TILE_ROWS * IW
N_ACC = N + 8
ROWS_PT = 624
ROWS_TAIL = N - NS * ROWS_PT

_PREC = lax.Precision.HIGHEST


def _sc_agg_body(vs_hbm, es_hbm, src_hbm, dst_hbm, zero_hbm, out_hbm,
                 sidx, didx, g0, g1, eb0, eb1, acc,
                 sg0, sg1, se0, se1):
    cid = lax.axis_index("c")
    sid = lax.axis_index("s")

    pltpu.sync_copy(zero_hbm.at[pl.ds(sid * ROWS_PT, ROWS_PT)],
                    acc.at[pl.ds(sid * ROWS_PT, ROWS_PT)])

    @pl.when(sid == 0)
    def _():
        tail = N_ACC - NS * ROWS_PT
        pltpu.sync_copy(zero_hbm.at[pl.ds(NS * ROWS_PT, tail)],
                        acc.at[pl.ds(NS * ROWS_PT, tail)])

    pltpu.sync_copy(src_hbm.at[pl.ds(sid * TILE_ROWS, TILE_ROWS)], sidx)
    pltpu.sync_copy(dst_hbm.at[pl.ds(sid * TILE_ROWS, TILE_ROWS)], didx)
    plsc.subcore_barrier()

    vtab = vs_hbm.at[cid]

    def gather_cp(j, gb, sg):
        return pltpu.make_async_copy(vtab.at[sidx.at[j]], gb, sg)

    def e_cp(j, ebuf, se):
        base = (sid * TILE_ROWS + j) * IW
        return pltpu.make_async_copy(es_hbm.at[cid, pl.ds(base, IW)], ebuf, se)

    def compute(gb, ebuf):
        @pl.loop(0, IW)
        def _row(r):
            for c16 in range(HH // 16):
                slc = (pl.ds(r, 1), pl.ds(c16 * 16, 16))
                gb.at[slc][...] = (
                    jnp.maximum(gb.at[slc][...] + ebuf.at[slc][...], 0.0)
                    + EPS)

    def step(j, gb, ebuf, sg, se, prefetch_j, pg, peb, psg, pse):
        @pl.when(prefetch_j < TILE_ROWS)
        def _():
            gather_cp(prefetch_j, pg, psg).start()
            e_cp(prefetch_j, peb, pse).start()

        gather_cp(j, gb, sg).wait()
        e_cp(j, ebuf, se).wait()
        compute(gb, ebuf)
        pltpu.sync_copy(gb, acc.at[didx.at[j]], add=True)

    gather_cp(0, g0, sg0).start()
    e_cp(0, eb0, se0).start()

    @pl.loop(0, TILE_ROWS // 2)
    def _pair(i):
        j0 = 2 * i
        step(j0, g0, eb0, sg0, se0, j0 + 1, g1, eb1, sg1, se1)
        step(j0 + 1, g1, eb1, sg1, se1, j0 + 2, g0, eb0, sg0, se0)

    plsc.subcore_barrier()
    pltpu.sync_copy(acc.at[pl.ds(sid * ROWS_PT, ROWS_PT)],
                    out_hbm.at[cid, pl.ds(sid * ROWS_PT, ROWS_PT)])

    @pl.when(sid == 0)
    def _():
        pltpu.sync_copy(acc.at[pl.ds(NS * ROWS_PT, ROWS_TAIL)],
                        out_hbm.at[cid, pl.ds(NS * ROWS_PT, ROWS_TAIL)])


def _sc_agg(vs, es, src2, dst2, zero):
    mesh = plsc.VectorSubcoreMesh(core_axis_name="c", subcore_axis_name="s")
    k = pl.kernel(
        _sc_agg_body,
        mesh=mesh,
        out_type=jax.ShapeDtypeStruct((NC, N, HH), jnp.float32),
        scratch_types=[
            pltpu.VMEM((TILE_ROWS, IW), jnp.int32),
            pltpu.VMEM((TILE_ROWS, IW), jnp.int32),
            pltpu.VMEM((IW, HH), jnp.float32),
            pltpu.VMEM((IW, HH), jnp.float32),
            pltpu.VMEM((IW, HH), jnp.float32),
            pltpu.VMEM((IW, HH), jnp.float32),
            pltpu.VMEM_SHARED((N_ACC, HH), jnp.float32),
            pltpu.SemaphoreType.DMA,
            pltpu.SemaphoreType.DMA,
            pltpu.SemaphoreType.DMA,
            pltpu.SemaphoreType.DMA,
        ],
        compiler_params=pltpu.CompilerParams(use_tc_tiling_on_sc=False),
    )
    return k(vs, es, src2, dst2, zero)


def _enc_body(x_ref, w1_ref, b1_ref, w2_ref, b2_ref, o_ref):
    h = jnp.maximum(
        jnp.dot(x_ref[...], w1_ref[...],
                preferred_element_type=jnp.float32, precision=_PREC)
        + b1_ref[...], 0.0)
    y = (jnp.dot(h, w2_ref[...],
                 preferred_element_type=jnp.float32, precision=_PREC)
         + b2_ref[...])
    o_ref[0] = y[:, :HH]
    o_ref[1] = y[:, HH:]


def _encode(x, p, bm, rows_out):
    W1, b1, W2, b2 = p
    M, din = x.shape
    dh = W1.shape[1]
    return pl.pallas_call(
        _enc_body,
        grid=(M // bm,),
        in_specs=[
            pl.BlockSpec((bm, din), lambda i: (i, 0)),
            pl.BlockSpec((din, dh), lambda i: (0, 0)),
            pl.BlockSpec((1, dh), lambda i: (0, 0)),
            pl.BlockSpec((dh, H), lambda i: (0, 0)),
            pl.BlockSpec((1, H), lambda i: (0, 0)),
        ],
        out_specs=pl.BlockSpec((2, bm, HH), lambda i: (0, i, 0)),
        out_shape=jax.ShapeDtypeStruct((2, rows_out, HH), jnp.float32),
    )(x, W1, b1.reshape(1, dh), W2, b2.reshape(1, H))


def _layer_body(a_ref, v_ref, w1_ref, b1_ref, w2_ref, b2_ref, o_ref):
    w1 = w1_ref[...]
    h = jnp.maximum(
        jnp.dot(a_ref[0], w1[:HH, :],
                preferred_element_type=jnp.float32, precision=_PREC)
        + jnp.dot(a_ref[1], w1[HH:, :],
                  preferred_element_type=jnp.float32, precision=_PREC)
        + b1_ref[...], 0.0)
    y = (jnp.dot(h, w2_ref[...],
                 preferred_element_type=jnp.float32, precision=_PREC)
         + b2_ref[...])
    o_ref[0] = v_ref[0] + y[:, :HH]
    o_ref[1] = v_ref[1] + y[:, HH:]


def _layer_mlp(aggs, vs, p, bm):
    W1, b1, W2, b2 = p
    return pl.pallas_call(
        _layer_body,
        grid=(N // bm,),
        in_specs=[
            pl.BlockSpec((2, bm, HH), lambda i: (0, i, 0)),
            pl.BlockSpec((2, bm, HH), lambda i: (0, i, 0)),
            pl.BlockSpec((H, H), lambda i: (0, 0)),
            pl.BlockSpec((1, H), lambda i: (0, 0)),
            pl.BlockSpec((H, H), lambda i: (0, 0)),
            pl.BlockSpec((1, H), lambda i: (0, 0)),
        ],
        out_specs=pl.BlockSpec((2, bm, HH), lambda i: (0, i, 0)),
        out_shape=jax.ShapeDtypeStruct((2, N, HH), jnp.float32),
    )(aggs, vs, W1, b1.reshape(1, H), W2, b2.reshape(1, H))


def _dec_body(x_ref, w1_ref, b1_ref, w2_ref, b2_ref, o_ref):
    w1 = w1_ref[...]
    h = jnp.maximum(
        jnp.dot(x_ref[0], w1[:HH, :],
                preferred_element_type=jnp.float32, precision=_PREC)
        + jnp.dot(x_ref[1], w1[HH:, :],
                  preferred_element_type=jnp.float32, precision=_PREC)
        + b1_ref[...], 0.0)
    o_ref[...] = (
        jnp.dot(h, w2_ref[...],
                preferred_element_type=jnp.float32, precision=_PREC)
        + b2_ref[...])


def _decode(xs, p, bm, rows):
    W1, b1, W2, b2 = p
    dh = W1.shape[1]
    dout = W2.shape[1]
    dpad = 8
    W2p = jnp.zeros((dh, dpad), jnp.float32).at[:, :dout].set(W2)
    b2p = jnp.zeros((1, dpad), jnp.float32).at[0, :dout].set(b2)
    y = pl.pallas_call(
        _dec_body,
        grid=(rows // bm,),
        in_specs=[
            pl.BlockSpec((2, bm, HH), lambda i: (0, i, 0)),
            pl.BlockSpec((H, dh), lambda i: (0, 0)),
            pl.BlockSpec((1, dh), lambda i: (0, 0)),
            pl.BlockSpec((dh, dpad), lambda i: (0, 0)),
            pl.BlockSpec((1, dpad), lambda i: (0, 0)),
        ],
        out_specs=pl.BlockSpec((bm, dpad), lambda i: (i, 0)),
        out_shape=jax.ShapeDtypeStruct((rows, dpad), jnp.float32),
    )(xs, W1, b1.reshape(1, dh), W2p, b2p)
    return y[:, :dout]


def kernel(v_x, v_edge_index, e_x, e_edge_index, enc_v, enc_e, layers,
           dec_v, dec_e):
    src = v_edge_index[0]
    dst = v_edge_index[1]
    src2 = jnp.zeros((E_PAD,), jnp.int32).at[:E].set(src).reshape(E_PAD // IW, IW)
    dst2 = jnp.full((E_PAD,), N, jnp.int32).at[:E].set(dst).reshape(E_PAD // IW, IW)
    zero = jnp.zeros((N_ACC, HH), jnp.float32)

    vs = _encode(v_x, enc_v, bm=1000, rows_out=N)
    es = _encode(e_x, enc_e, bm=2000, rows_out=E_PAD)

    for (pv, _pe) in layers:
        aggs = _sc_agg(vs, es, src2, dst2, zero)
        vs = _layer_mlp(aggs, vs, pv, bm=1000)

    node_out = _decode(vs, dec_v, bm=1000, rows=N)
    edge_out = _decode(es, dec_e, bm=2000, rows=E)
    return (node_out, edge_out)


# speedup vs baseline: 2.5122x; 1.1861x over previous
import jax
import jax.numpy as jnp
from jax import lax
from jax.experimental import pallas as pl
from jax.experimental.pallas import tpu as pltpu
from jax.experimental.pallas import tpu_sc as plsc

N = 10000
E = 320000
H = 128
HH = H // 2
EPS = 1e-7

NC = 2
NS = 16
IW = 128
TILE_ROWS = 160
E_PAD = NS * TILE_ROWS * IW
N_ACC = N + 8
ROWS_PT = 624
ROWS_TAIL = N - NS * ROWS_PT

def _dot3(x, w):
    xh = x.astype(jnp.bfloat16)
    xl = (x - xh.astype(jnp.float32)).astype(jnp.bfloat16)
    wh = w.astype(jnp.bfloat16)
    wl = (w - wh.astype(jnp.float32)).astype(jnp.bfloat16)
    acc = jnp.dot(xh, wl, preferred_element_type=jnp.float32)
    acc += jnp.dot(xl, wh, preferred_element_type=jnp.float32)
    acc += jnp.dot(xh, wh, preferred_element_type=jnp.float32)
    return acc


def _sc_agg_body(vs_hbm, es_hbm, src_hbm, dst_hbm, zero_hbm, out_hbm,
                 sidx, didx, g0, g1, eb0, eb1, acc,
                 sg0, sg1, se0, se1):
    cid = lax.axis_index("c")
    sid = lax.axis_index("s")

    pltpu.sync_copy(zero_hbm.at[pl.ds(sid * ROWS_PT, ROWS_PT)],
                    acc.at[pl.ds(sid * ROWS_PT, ROWS_PT)])

    @pl.when(sid == 0)
    def _():
        tail = N_ACC - NS * ROWS_PT
        pltpu.sync_copy(zero_hbm.at[pl.ds(NS * ROWS_PT, tail)],
                        acc.at[pl.ds(NS * ROWS_PT, tail)])

    pltpu.sync_copy(src_hbm.at[pl.ds(sid * TILE_ROWS, TILE_ROWS)], sidx)
    pltpu.sync_copy(dst_hbm.at[pl.ds(sid * TILE_ROWS, TILE_ROWS)], didx)
    plsc.subcore_barrier()

    vtab = vs_hbm.at[cid]

    def gather_cp(j, gb, sg):
        return pltpu.make_async_copy(vtab.at[sidx.at[j]], gb, sg)

    def e_cp(j, ebuf, se):
        base = (sid * TILE_ROWS + j) * IW
        return pltpu.make_async_copy(es_hbm.at[cid, pl.ds(base, IW)], ebuf, se)

    def compute(gb, ebuf):
        @pl.loop(0, IW)
        def _row(r):
            for c16 in range(HH // 16):
                slc = (pl.ds(r, 1), pl.ds(c16 * 16, 16))
                gb.at[slc][...] = (
                    jnp.maximum(gb.at[slc][...] + ebuf.at[slc][...], 0.0)
                    + EPS)

    def step(j, gb, ebuf, sg, se, prefetch_j, pg, peb, psg, pse):
        @pl.when(prefetch_j < TILE_ROWS)
        def _():
            gather_cp(prefetch_j, pg, psg).start()
            e_cp(prefetch_j, peb, pse).start()

        gather_cp(j, gb, sg).wait()
        e_cp(j, ebuf, se).wait()
        compute(gb, ebuf)
        pltpu.sync_copy(gb, acc.at[didx.at[j]], add=True)

    gather_cp(0, g0, sg0).start()
    e_cp(0, eb0, se0).start()

    @pl.loop(0, TILE_ROWS // 2)
    def _pair(i):
        j0 = 2 * i
        step(j0, g0, eb0, sg0, se0, j0 + 1, g1, eb1, sg1, se1)
        step(j0 + 1, g1, eb1, sg1, se1, j0 + 2, g0, eb0, sg0, se0)

    plsc.subcore_barrier()
    pltpu.sync_copy(acc.at[pl.ds(sid * ROWS_PT, ROWS_PT)],
                    out_hbm.at[cid, pl.ds(sid * ROWS_PT, ROWS_PT)])

    @pl.when(sid == 0)
    def _():
        pltpu.sync_copy(acc.at[pl.ds(NS * ROWS_PT, ROWS_TAIL)],
                        out_hbm.at[cid, pl.ds(NS * ROWS_PT, ROWS_TAIL)])


def _sc_agg(vs, es, src2, dst2, zero):
    mesh = plsc.VectorSubcoreMesh(core_axis_name="c", subcore_axis_name="s")
    k = pl.kernel(
        _sc_agg_body,
        mesh=mesh,
        out_type=jax.ShapeDtypeStruct((NC, N, HH), jnp.float32),
        scratch_types=[
            pltpu.VMEM((TILE_ROWS, IW), jnp.int32),
            pltpu.VMEM((TILE_ROWS, IW), jnp.int32),
            pltpu.VMEM((IW, HH), jnp.float32),
            pltpu.VMEM((IW, HH), jnp.float32),
            pltpu.VMEM((IW, HH), jnp.float32),
            pltpu.VMEM((IW, HH), jnp.float32),
            pltpu.VMEM_SHARED((N_ACC, HH), jnp.float32),
            pltpu.SemaphoreType.DMA,
            pltpu.SemaphoreType.DMA,
            pltpu.SemaphoreType.DMA,
            pltpu.SemaphoreType.DMA,
        ],
        compiler_params=pltpu.CompilerParams(use_tc_tiling_on_sc=False),
    )
    return k(vs, es, src2, dst2, zero)


def _enc_body(x_ref, w1_ref, b1_ref, w2_ref, b2_ref, o_ref):
    h = jnp.maximum(
        _dot3(x_ref[...], w1_ref[...])
        + b1_ref[...], 0.0)
    y = (_dot3(h, w2_ref[...])
         + b2_ref[...])
    o_ref[0] = y[:, :HH]
    o_ref[1] = y[:, HH:]


def _encode(x, p, bm, rows_out):
    W1, b1, W2, b2 = p
    M, din = x.shape
    dh = W1.shape[1]
    return pl.pallas_call(
        _enc_body,
        grid=(M // bm,),
        in_specs=[
            pl.BlockSpec((bm, din), lambda i: (i, 0)),
            pl.BlockSpec((din, dh), lambda i: (0, 0)),
            pl.BlockSpec((1, dh), lambda i: (0, 0)),
            pl.BlockSpec((dh, H), lambda i: (0, 0)),
            pl.BlockSpec((1, H), lambda i: (0, 0)),
        ],
        out_specs=pl.BlockSpec((2, bm, HH), lambda i: (0, i, 0)),
        out_shape=jax.ShapeDtypeStruct((2, rows_out, HH), jnp.float32),
    )(x, W1, b1.reshape(1, dh), W2, b2.reshape(1, H))


def _layer_body(a_ref, v_ref, w1_ref, b1_ref, w2_ref, b2_ref, o_ref):
    w1 = w1_ref[...]
    h = jnp.maximum(
        _dot3(a_ref[0], w1[:HH, :])
        + _dot3(a_ref[1], w1[HH:, :])
        + b1_ref[...], 0.0)
    y = (_dot3(h, w2_ref[...])
         + b2_ref[...])
    o_ref[0] = v_ref[0] + y[:, :HH]
    o_ref[1] = v_ref[1] + y[:, HH:]


def _layer_mlp(aggs, vs, p, bm):
    W1, b1, W2, b2 = p
    return pl.pallas_call(
        _layer_body,
        grid=(N // bm,),
        in_specs=[
            pl.BlockSpec((2, bm, HH), lambda i: (0, i, 0)),
            pl.BlockSpec((2, bm, HH), lambda i: (0, i, 0)),
            pl.BlockSpec((H, H), lambda i: (0, 0)),
            pl.BlockSpec((1, H), lambda i: (0, 0)),
            pl.BlockSpec((H, H), lambda i: (0, 0)),
            pl.BlockSpec((1, H), lambda i: (0, 0)),
        ],
        out_specs=pl.BlockSpec((2, bm, HH), lambda i: (0, i, 0)),
        out_shape=jax.ShapeDtypeStruct((2, N, HH), jnp.float32),
    )(aggs, vs, W1, b1.reshape(1, H), W2, b2.reshape(1, H))


def _dec_body(x_ref, w1_ref, b1_ref, w2_ref, b2_ref, o_ref):
    w1 = w1_ref[...]
    h = jnp.maximum(
        _dot3(x_ref[0], w1[:HH, :])
        + _dot3(x_ref[1], w1[HH:, :])
        + b1_ref[...], 0.0)
    o_ref[...] = (
        _dot3(h, w2_ref[...])
        + b2_ref[...])


def _decode(xs, p, bm, rows):
    W1, b1, W2, b2 = p
    dh = W1.shape[1]
    dout = W2.shape[1]
    dpad = 8
    W2p = jnp.zeros((dh, dpad), jnp.float32).at[:, :dout].set(W2)
    b2p = jnp.zeros((1, dpad), jnp.float32).at[0, :dout].set(b2)
    y = pl.pallas_call(
        _dec_body,
        grid=(rows // bm,),
        in_specs=[
            pl.BlockSpec((2, bm, HH), lambda i: (0, i, 0)),
            pl.BlockSpec((H, dh), lambda i: (0, 0)),
            pl.BlockSpec((1, dh), lambda i: (0, 0)),
            pl.BlockSpec((dh, dpad), lambda i: (0, 0)),
            pl.BlockSpec((1, dpad), lambda i: (0, 0)),
        ],
        out_specs=pl.BlockSpec((bm, dpad), lambda i: (i, 0)),
        out_shape=jax.ShapeDtypeStruct((rows, dpad), jnp.float32),
    )(xs, W1, b1.reshape(1, dh), W2p, b2p)
    return y[:, :dout]


def kernel(v_x, v_edge_index, e_x, e_edge_index, enc_v, enc_e, layers,
           dec_v, dec_e):
    src = v_edge_index[0]
    dst = v_edge_index[1]
    src2 = jnp.pad(src, (0, E_PAD - E)).reshape(E_PAD // IW, IW)
    dst2 = jnp.pad(dst, (0, E_PAD - E),
                   constant_values=N).reshape(E_PAD // IW, IW)
    zero = jnp.zeros((N_ACC, HH), jnp.float32)

    vs = _encode(v_x, enc_v, bm=1000, rows_out=N)
    es = _encode(e_x, enc_e, bm=2000, rows_out=E_PAD)
    edge_out = _decode(es, dec_e, bm=2000, rows=E)

    for (pv, _pe) in layers:
        aggs = _sc_agg(vs, es, src2, dst2, zero)
        vs = _layer_mlp(aggs, vs, pv, bm=1000)

    node_out = _decode(vs, dec_v, bm=1000, rows=N)
    return (node_out, edge_out)


# speedup vs baseline: 3.8609x; 1.5369x over previous
import jax
import jax.numpy as jnp
from jax import lax
from jax.experimental import pallas as pl
from jax.experimental.pallas import tpu as pltpu
from jax.experimental.pallas import tpu_sc as plsc

N = 10000
E = 320000
H = 128
HH = H // 2
EPS = 1e-7

NC = 2
NS = 16
IW = 128
EPT = E // NS
FULL_STEPS = EPT // IW
TAIL = EPT - FULL_STEPS * IW
ROWS_PT = 624
ROWS_TAIL = N - NS * ROWS_PT

def _dot3(x, w):
    return jnp.dot(x.astype(jnp.bfloat16), w.astype(jnp.bfloat16),
                   preferred_element_type=jnp.float32)


def _dotx(x, w):
    xh = x.astype(jnp.bfloat16)
    xl = (x - xh.astype(jnp.float32)).astype(jnp.bfloat16)
    wh = w.astype(jnp.bfloat16)
    wl = (w - wh.astype(jnp.float32)).astype(jnp.bfloat16)
    acc = jnp.dot(xh, wl, preferred_element_type=jnp.float32)
    acc += jnp.dot(xl, wh, preferred_element_type=jnp.float32)
    acc += jnp.dot(xh, wh, preferred_element_type=jnp.float32)
    return acc


def _sc_agg_body(vs_hbm, es_hbm, vei_hbm, zero_hbm, out_hbm,
                 sidx, didx, g0, g1, eb0, eb1, acc,
                 sg0, sg1, se0, se1, ss0, ss1):
    cid = lax.axis_index("c")
    sid = lax.axis_index("s")

    pltpu.sync_copy(zero_hbm.at[pl.ds(sid * ROWS_PT, ROWS_PT)],
                    acc.at[pl.ds(sid * ROWS_PT, ROWS_PT)])

    @pl.when(sid == 0)
    def _():
        pltpu.sync_copy(zero_hbm.at[pl.ds(NS * ROWS_PT, ROWS_TAIL)],
                        acc.at[pl.ds(NS * ROWS_PT, ROWS_TAIL)])

    base0 = sid * EPT
    pltpu.sync_copy(vei_hbm.at[0, pl.ds(base0, EPT)], sidx)
    pltpu.sync_copy(vei_hbm.at[1, pl.ds(base0, EPT)], didx)
    plsc.subcore_barrier()

    vtab = vs_hbm.at[cid]

    def gather_start(j, w, gb, sg):
        pltpu.async_copy(vtab.at[sidx.at[pl.ds(j * IW, w)]], gb, sg)

    def gather_wait(j, w, gb, sg):
        pltpu.make_async_copy(
            vtab.at[sidx.at[pl.ds(j * IW, w)]], gb, sg).wait()

    def e_start(j, w, ebuf, se):
        pltpu.async_copy(
            es_hbm.at[cid, pl.ds(base0 + j * IW, w)], ebuf, se)

    def e_wait(j, w, ebuf, se):
        pltpu.make_async_copy(
            es_hbm.at[cid, pl.ds(base0 + j * IW, w)], ebuf, se).wait()

    def scatter_start(j, w, gb, ss):
        pltpu.async_copy(gb, acc.at[didx.at[pl.ds(j * IW, w)]], ss, add=True)

    def scatter_wait(j, w, gb, ss):
        pltpu.make_async_copy(
            gb, acc.at[didx.at[pl.ds(j * IW, w)]], ss).wait()

    def compute(gb, ebuf, rows):
        @pl.loop(0, rows, step=4)
        def _row(r):
            for r4 in range(4):
                for c16 in range(HH // 16):
                    slc = (pl.ds(r + r4, 1), pl.ds(c16 * 16, 16))
                    gb.at[slc][...] = (
                        jnp.maximum(gb.at[slc][...] + ebuf.at[slc][...], 0.0)
                        + EPS)

    def step(j, gb, ebuf, sg, se, ss, og, oss):
        pj = j + 1

        @pl.when(pj < FULL_STEPS)
        def _():
            @pl.when(j >= 1)
            def _():
                scatter_wait(j - 1, IW, og, oss)
            if ss is ss0:
                gather_start(pj, IW, g1, sg1)
                e_start(pj, IW, eb1, se1)
            else:
                gather_start(pj, IW, g0, sg0)
                e_start(pj, IW, eb0, se0)

        gather_wait(j, IW, gb, sg)
        e_wait(j, IW, ebuf, se)
        compute(gb, ebuf, IW)
        scatter_start(j, IW, gb, ss)

    gather_start(0, IW, g0, sg0)
    e_start(0, IW, eb0, se0)

    @pl.loop(0, FULL_STEPS // 2)
    def _pair(i):
        j0 = 2 * i
        step(j0, g0, eb0, sg0, se0, ss0, g1, ss1)
        step(j0 + 1, g1, eb1, sg1, se1, ss1, g0, ss0)

    scatter_wait(FULL_STEPS - 2, IW, g0, ss0)
    scatter_wait(FULL_STEPS - 1, IW, g1, ss1)

    gt = g0.at[pl.ds(0, TAIL)]
    et = eb0.at[pl.ds(0, TAIL)]
    pltpu.async_copy(vtab.at[sidx.at[pl.ds(FULL_STEPS * IW, TAIL)]],
                     gt, sg0).wait()
    pltpu.async_copy(es_hbm.at[cid, pl.ds(base0 + FULL_STEPS * IW, TAIL)],
                     et, se0).wait()
    compute(g0, eb0, TAIL)
    pltpu.sync_copy(gt, acc.at[didx.at[pl.ds(FULL_STEPS * IW, TAIL)]],
                    add=True)

    plsc.subcore_barrier()
    pltpu.sync_copy(acc.at[pl.ds(sid * ROWS_PT, ROWS_PT)],
                    out_hbm.at[cid, pl.ds(sid * ROWS_PT, ROWS_PT)])

    @pl.when(sid == 0)
    def _():
        pltpu.sync_copy(acc.at[pl.ds(NS * ROWS_PT, ROWS_TAIL)],
                        out_hbm.at[cid, pl.ds(NS * ROWS_PT, ROWS_TAIL)])


def _sc_agg(vs, es, vei, zero):
    mesh = plsc.VectorSubcoreMesh(core_axis_name="c", subcore_axis_name="s")
    k = pl.kernel(
        _sc_agg_body,
        mesh=mesh,
        out_type=jax.ShapeDtypeStruct((NC, N, HH), jnp.float32),
        scratch_types=[
            pltpu.VMEM((EPT,), jnp.int32),
            pltpu.VMEM((EPT,), jnp.int32),
            pltpu.VMEM((IW, HH), jnp.float32),
            pltpu.VMEM((IW, HH), jnp.float32),
            pltpu.VMEM((IW, HH), jnp.float32),
            pltpu.VMEM((IW, HH), jnp.float32),
            pltpu.VMEM_SHARED((N, HH), jnp.float32),
            pltpu.SemaphoreType.DMA,
            pltpu.SemaphoreType.DMA,
            pltpu.SemaphoreType.DMA,
            pltpu.SemaphoreType.DMA,
            pltpu.SemaphoreType.DMA,
            pltpu.SemaphoreType.DMA,
        ],
        compiler_params=pltpu.CompilerParams(use_tc_tiling_on_sc=False),
    )
    return k(vs, es, vei, zero)


def _enc_body(x_ref, w1_ref, b1_ref, w2_ref, b2_ref, o_ref, *, d1, d2):
    h = jnp.maximum(
        d1(x_ref[...], w1_ref[...])
        + b1_ref[...], 0.0)
    y = (d2(h, w2_ref[...])
         + b2_ref[...])
    o_ref[0] = y[:, :HH]
    o_ref[1] = y[:, HH:]


def _encode(x, p, bm, rows_out, dd=(_dot3, _dot3)):
    W1, b1, W2, b2 = p
    M, din = x.shape
    dh = W1.shape[1]
    import functools as _ft
    return pl.pallas_call(
        _ft.partial(_enc_body, d1=dd[0], d2=dd[1]),
        grid=(M // bm,),
        in_specs=[
            pl.BlockSpec((bm, din), lambda i: (i, 0)),
            pl.BlockSpec((din, dh), lambda i: (0, 0)),
            pl.BlockSpec((1, dh), lambda i: (0, 0)),
            pl.BlockSpec((dh, H), lambda i: (0, 0)),
            pl.BlockSpec((1, H), lambda i: (0, 0)),
        ],
        out_specs=pl.BlockSpec((2, bm, HH), lambda i: (0, i, 0)),
        out_shape=jax.ShapeDtypeStruct((2, rows_out, HH), jnp.float32),
    )(x, W1, b1.reshape(1, dh), W2, b2.reshape(1, H))


def _layer_body(a_ref, v_ref, w1_ref, b1_ref, w2_ref, b2_ref, o_ref, *, dd=None):
    d1, d2 = dd
    x = jnp.concatenate([a_ref[0], a_ref[1]], axis=1)
    h = jnp.maximum(d1(x, w1_ref[...]) + b1_ref[...], 0.0)
    y = (d2(h, w2_ref[...])
         + b2_ref[...])
    o_ref[0] = v_ref[0] + y[:, :HH]
    o_ref[1] = v_ref[1] + y[:, HH:]


def _layer_mlp(aggs, vs, p, bm, dd=(_dot3, _dot3)):
    W1, b1, W2, b2 = p
    import functools as _ft
    return pl.pallas_call(
        _ft.partial(_layer_body, dd=dd),
        grid=(N // bm,),
        in_specs=[
            pl.BlockSpec((2, bm, HH), lambda i: (0, i, 0)),
            pl.BlockSpec((2, bm, HH), lambda i: (0, i, 0)),
            pl.BlockSpec((H, H), lambda i: (0, 0)),
            pl.BlockSpec((1, H), lambda i: (0, 0)),
            pl.BlockSpec((H, H), lambda i: (0, 0)),
            pl.BlockSpec((1, H), lambda i: (0, 0)),
        ],
        out_specs=pl.BlockSpec((2, bm, HH), lambda i: (0, i, 0)),
        out_shape=jax.ShapeDtypeStruct((2, N, HH), jnp.float32),
    )(aggs, vs, W1, b1.reshape(1, H), W2, b2.reshape(1, H))


def _dec_body(x_ref, w1_ref, b1_ref, w2_ref, b2_ref, o_ref, *, dd=None):
    d1, d2 = dd
    x = jnp.concatenate([x_ref[0], x_ref[1]], axis=1)
    h = jnp.maximum(d1(x, w1_ref[...]) + b1_ref[...], 0.0)
    o_ref[...] = (
        d2(h, w2_ref[...])
        + b2_ref[...])


def _dec_dep_body(x_ref, w1_ref, b1_ref, w2_ref, b2_ref, dep_ref, o_ref, *, dd=None):
    del dep_ref
    _dec_body(x_ref, w1_ref, b1_ref, w2_ref, b2_ref, o_ref, dd=dd)


def _decode_chunk(xs, p, bm, rows, row0, dep):
    W1, b1, W2, b2 = p
    dh = W1.shape[1]
    dout = W2.shape[1]
    dpad = 8
    W2p = jnp.zeros((dh, dpad), jnp.float32).at[:, :dout].set(W2)
    b2p = jnp.zeros((1, dpad), jnp.float32).at[0, :dout].set(b2)
    blk0 = row0 // bm
    import functools as _ft
    y = pl.pallas_call(
        _ft.partial(_dec_dep_body, dd=(_dot3, _dot3)),
        grid=(rows // bm,),
        in_specs=[
            pl.BlockSpec((2, bm, HH), lambda i: (0, i + blk0, 0)),
            pl.BlockSpec((H, dh), lambda i: (0, 0)),
            pl.BlockSpec((1, dh), lambda i: (0, 0)),
            pl.BlockSpec((dh, dpad), lambda i: (0, 0)),
            pl.BlockSpec((1, dpad), lambda i: (0, 0)),
            pl.BlockSpec((1, 8, HH), lambda i: (0, 0, 0)),
        ],
        out_specs=pl.BlockSpec((bm, dpad), lambda i: (i, 0)),
        out_shape=jax.ShapeDtypeStruct((rows, dpad), jnp.float32),
    )(xs, W1, b1.reshape(1, dh), W2p, b2p, dep)
    return y[:, :dout]


def _decode(xs, p, bm, rows, dd=(_dot3, _dot3)):
    W1, b1, W2, b2 = p
    dh = W1.shape[1]
    dout = W2.shape[1]
    dpad = 8
    W2p = jnp.zeros((dh, dpad), jnp.float32).at[:, :dout].set(W2)
    b2p = jnp.zeros((1, dpad), jnp.float32).at[0, :dout].set(b2)
    import functools as _ft
    y = pl.pallas_call(
        _ft.partial(_dec_body, dd=dd),
        grid=(rows // bm,),
        in_specs=[
            pl.BlockSpec((2, bm, HH), lambda i: (0, i, 0)),
            pl.BlockSpec((H, dh), lambda i: (0, 0)),
            pl.BlockSpec((1, dh), lambda i: (0, 0)),
            pl.BlockSpec((dh, dpad), lambda i: (0, 0)),
            pl.BlockSpec((1, dpad), lambda i: (0, 0)),
        ],
        out_specs=pl.BlockSpec((bm, dpad), lambda i: (i, 0)),
        out_shape=jax.ShapeDtypeStruct((rows, dpad), jnp.float32),
    )(xs, W1, b1.reshape(1, dh), W2p, b2p)
    return y[:, :dout]


def kernel(v_x, v_edge_index, e_x, e_edge_index, enc_v, enc_e, layers,
           dec_v, dec_e):
    zero = jnp.zeros((N, HH), jnp.float32)

    _XX = (_dot3, _dot3)
    vs = _encode(v_x, enc_v, bm=1000, rows_out=N, dd=_XX)
    es = _encode(e_x, enc_e, bm=2000, rows_out=E)

    chunk_dep_layer = [0, 1, 2, 2]
    chunk_rows = E // 4
    aggs_hist = []
    edge_chunks = []
    for (pv, _pe) in layers:
        aggs = _sc_agg(vs, es, v_edge_index, zero)
        aggs_hist.append(aggs)
        vs = _layer_mlp(aggs, vs, pv, bm=1000, dd=_XX)
    for k in range(4):
        edge_chunks.append(
            _decode_chunk(es, dec_e, bm=2000, rows=chunk_rows,
                          row0=k * chunk_rows,
                          dep=aggs_hist[chunk_dep_layer[k]]))
    edge_out = jnp.concatenate(edge_chunks, axis=0)

    node_out = _decode(vs, dec_v, bm=1000, rows=N, dd=_XX)
    return (node_out, edge_out)


# speedup vs baseline: 4.9470x; 1.2813x over previous
import jax
import jax.numpy as jnp
from jax import lax
from jax.experimental import pallas as pl
from jax.experimental.pallas import tpu as pltpu
from jax.experimental.pallas import tpu_sc as plsc

N = 10000
E = 320000
H = 128
EPS = 1e-7

NC = 2
NS = 16
IW = 64
EPC = E // NC
EPT = EPC // NS
FULL_STEPS = EPT // IW
TAIL = EPT - FULL_STEPS * IW
ROWS_PT = 624
ROWS_TAIL = N - NS * ROWS_PT


def _dot3(x, w):
    return jnp.dot(x.astype(jnp.bfloat16), w.astype(jnp.bfloat16),
                   preferred_element_type=jnp.float32)


def _sc_agg_body(v_hbm, e_hbm, src_hbm, dst_hbm, zero_hbm, out_hbm,
                 sidx, di0, di1, g0, g1, eb0, eb1, acc,
                 sg0, sg1, se0, se1, ss0, ss1, sd0, sd1):
    cid = lax.axis_index("c")
    sid = lax.axis_index("s")

    pltpu.sync_copy(zero_hbm.at[pl.ds(sid * ROWS_PT, ROWS_PT)],
                    acc.at[pl.ds(sid * ROWS_PT, ROWS_PT)])

    @pl.when(sid == 0)
    def _():
        pltpu.sync_copy(zero_hbm.at[pl.ds(NS * ROWS_PT, ROWS_TAIL)],
                        acc.at[pl.ds(NS * ROWS_PT, ROWS_TAIL)])

    base0 = cid * EPC + sid * EPT
    pltpu.sync_copy(src_hbm.at[pl.ds(base0, EPT)], sidx)
    plsc.subcore_barrier()

    def gather_start(j, w, gb, sg):
        pltpu.async_copy(v_hbm.at[sidx.at[pl.ds(j * IW, w)]], gb, sg)

    def gather_wait(j, w, gb, sg):
        pltpu.make_async_copy(
            v_hbm.at[sidx.at[pl.ds(j * IW, w)]], gb, sg).wait()

    def e_start(j, w, ebuf, se):
        pltpu.async_copy(e_hbm.at[pl.ds(base0 + j * IW, w)], ebuf, se)

    def e_wait(j, w, ebuf, se):
        pltpu.make_async_copy(
            e_hbm.at[pl.ds(base0 + j * IW, w)], ebuf, se).wait()

    def d_start(j, w, dib, sd):
        pltpu.async_copy(dst_hbm.at[pl.ds(base0 + j * IW, w)], dib, sd)

    def d_wait(j, w, dib, sd):
        pltpu.make_async_copy(
            dst_hbm.at[pl.ds(base0 + j * IW, w)], dib, sd).wait()

    def scatter_start(gb, dib, ss):
        pltpu.async_copy(gb, acc.at[dib], ss, add=True)

    def scatter_wait(gb, dib, ss):
        pltpu.make_async_copy(gb, acc.at[dib], ss).wait()

    def compute(gb, ebuf, rows):
        @pl.loop(0, rows, step=2)
        def _row(r):
            for r2 in range(2):
                for c16 in range(H // 16):
                    slc = (pl.ds(r + r2, 1), pl.ds(c16 * 16, 16))
                    gb.at[slc][...] = (
                        jnp.maximum(gb.at[slc][...] + ebuf.at[slc][...], 0.0)
                        + EPS)

    def step(j, gb, ebuf, dib, sg, se, sd, ss, og, odi, oss):
        pj = j + 1

        @pl.when(pj < FULL_STEPS)
        def _():
            @pl.when(j >= 1)
            def _():
                scatter_wait(og, odi, oss)
            if ss is ss0:
                gather_start(pj, IW, g1, sg1)
                e_start(pj, IW, eb1, se1)
                d_start(pj, IW, di1, sd1)
            else:
                gather_start(pj, IW, g0, sg0)
                e_start(pj, IW, eb0, se0)
                d_start(pj, IW, di0, sd0)

        gather_wait(j, IW, gb, sg)
        e_wait(j, IW, ebuf, se)
        compute(gb, ebuf, IW)
        d_wait(j, IW, dib, sd)
        scatter_start(gb, dib, ss)

    gather_start(0, IW, g0, sg0)
    e_start(0, IW, eb0, se0)
    d_start(0, IW, di0, sd0)

    @pl.loop(0, FULL_STEPS // 2)
    def _pair(i):
        j0 = 2 * i
        step(j0, g0, eb0, di0, sg0, se0, sd0, ss0, g1, di1, ss1)
        step(j0 + 1, g1, eb1, di1, sg1, se1, sd1, ss1, g0, di0, ss0)

    scatter_wait(g0, di0, ss0)
    scatter_wait(g1, di1, ss1)

    gt = g0.at[pl.ds(0, TAIL)]
    et = eb0.at[pl.ds(0, TAIL)]
    dt = di0.at[pl.ds(0, TAIL)]
    pltpu.async_copy(v_hbm.at[sidx.at[pl.ds(FULL_STEPS * IW, TAIL)]],
                     gt, sg0).wait()
    pltpu.async_copy(e_hbm.at[pl.ds(base0 + FULL_STEPS * IW, TAIL)],
                     et, se0).wait()
    pltpu.async_copy(dst_hbm.at[pl.ds(base0 + FULL_STEPS * IW, TAIL)],
                     dt, sd0).wait()
    compute(g0, eb0, TAIL)
    pltpu.sync_copy(gt, acc.at[dt], add=True)

    plsc.subcore_barrier()
    pltpu.sync_copy(acc.at[pl.ds(sid * ROWS_PT, ROWS_PT)],
                    out_hbm.at[cid, pl.ds(sid * ROWS_PT, ROWS_PT)])

    @pl.when(sid == 0)
    def _():
        pltpu.sync_copy(acc.at[pl.ds(NS * ROWS_PT, ROWS_TAIL)],
                        out_hbm.at[cid, pl.ds(NS * ROWS_PT, ROWS_TAIL)])


def _sc_agg(v, e, src, dst, zero):
    mesh = plsc.VectorSubcoreMesh(core_axis_name="c", subcore_axis_name="s")
    k = pl.kernel(
        _sc_agg_body,
        mesh=mesh,
        out_type=jax.ShapeDtypeStruct((NC, N, H), jnp.float32),
        scratch_types=[
            pltpu.VMEM((EPT,), jnp.int32),
            pltpu.VMEM((IW,), jnp.int32),
            pltpu.VMEM((IW,), jnp.int32),
            pltpu.VMEM((IW, H), jnp.float32),
            pltpu.VMEM((IW, H), jnp.float32),
            pltpu.VMEM((IW, H), jnp.float32),
            pltpu.VMEM((IW, H), jnp.float32),
            pltpu.VMEM_SHARED((N, H), jnp.float32),
            pltpu.SemaphoreType.DMA,
            pltpu.SemaphoreType.DMA,
            pltpu.SemaphoreType.DMA,
            pltpu.SemaphoreType.DMA,
            pltpu.SemaphoreType.DMA,
            pltpu.SemaphoreType.DMA,
            pltpu.SemaphoreType.DMA,
            pltpu.SemaphoreType.DMA,
        ],
    )
    return k(v, e, src, dst, zero)


def _mlp_body(x_ref, w1_ref, b1_ref, w2_ref, b2_ref, o_ref):
    h = jnp.maximum(_dot3(x_ref[...], w1_ref[...]) + b1_ref[...], 0.0)
    o_ref[...] = _dot3(h, w2_ref[...]) + b2_ref[...]


def _mlp(x, p, bm):
    W1, b1, W2, b2 = p
    M, din = x.shape
    dh = W1.shape[1]
    dout = W2.shape[1]
    return pl.pallas_call(
        _mlp_body,
        grid=(M // bm,),
        in_specs=[
            pl.BlockSpec((bm, din), lambda i: (i, 0)),
            pl.BlockSpec((din, dh), lambda i: (0, 0)),
            pl.BlockSpec((1, dh), lambda i: (0, 0)),
            pl.BlockSpec((dh, dout), lambda i: (0, 0)),
            pl.BlockSpec((1, dout), lambda i: (0, 0)),
        ],
        out_specs=pl.BlockSpec((bm, dout), lambda i: (i, 0)),
        out_shape=jax.ShapeDtypeStruct((M, dout), jnp.float32),
    )(x, W1, b1.reshape(1, dh), W2, b2.reshape(1, dout))


def _layer_body(a_ref, v_ref, w1_ref, b1_ref, w2_ref, b2_ref, o_ref):
    x = a_ref[0] + a_ref[1]
    h = jnp.maximum(_dot3(x, w1_ref[...]) + b1_ref[...], 0.0)
    o_ref[...] = v_ref[...] + _dot3(h, w2_ref[...]) + b2_ref[...]


def _layer_mlp(aggs, v, p, bm):
    W1, b1, W2, b2 = p
    return pl.pallas_call(
        _layer_body,
        grid=(N // bm,),
        in_specs=[
            pl.BlockSpec((2, bm, H), lambda i: (0, i, 0)),
            pl.BlockSpec((bm, H), lambda i: (i, 0)),
            pl.BlockSpec((H, H), lambda i: (0, 0)),
            pl.BlockSpec((1, H), lambda i: (0, 0)),
            pl.BlockSpec((H, H), lambda i: (0, 0)),
            pl.BlockSpec((1, H), lambda i: (0, 0)),
        ],
        out_specs=pl.BlockSpec((bm, H), lambda i: (i, 0)),
        out_shape=jax.ShapeDtypeStruct((N, H), jnp.float32),
    )(aggs, v, W1, b1.reshape(1, H), W2, b2.reshape(1, H))


def _pad_dec(p):
    W1, b1, W2, b2 = p
    dh, dout = W2.shape
    dpad = 8
    W2p = jnp.zeros((dh, dpad), jnp.float32).at[:, :dout].set(W2)
    b2p = jnp.zeros((dpad,), jnp.float32).at[:dout].set(b2)
    return (W1, b1, W2p, b2p), dout


def _dec_dep_body(x_ref, w1_ref, b1_ref, w2_ref, b2_ref, dep_ref, o_ref):
    del dep_ref
    _mlp_body(x_ref, w1_ref, b1_ref, w2_ref, b2_ref, o_ref)


def _decode_chunk(x, p, bm, rows, row0, dep):
    W1, b1, W2, b2 = p
    dh = W1.shape[1]
    dout = W2.shape[1]
    blk0 = row0 // bm
    return pl.pallas_call(
        _dec_dep_body,
        grid=(rows // bm,),
        in_specs=[
            pl.BlockSpec((bm, H), lambda i: (i + blk0, 0)),
            pl.BlockSpec((H, dh), lambda i: (0, 0)),
            pl.BlockSpec((1, dh), lambda i: (0, 0)),
            pl.BlockSpec((dh, dout), lambda i: (0, 0)),
            pl.BlockSpec((1, dout), lambda i: (0, 0)),
            pl.BlockSpec((1, 8, H), lambda i: (0, 0, 0)),
        ],
        out_specs=pl.BlockSpec((bm, dout), lambda i: (i, 0)),
        out_shape=jax.ShapeDtypeStruct((rows, dout), jnp.float32),
    )(x, W1, b1.reshape(1, dh), W2, b2.reshape(1, dout), dep)


def kernel(v_x, v_edge_index, e_x, e_edge_index, enc_v, enc_e, layers,
           dec_v, dec_e):
    src = v_edge_index[0]
    dst = v_edge_index[1]
    zero = jnp.zeros((N, H), jnp.float32)

    v = _mlp(v_x, enc_v, bm=1000)
    e = _mlp(e_x, enc_e, bm=2000)

    dec_e_p, dout_e = _pad_dec(dec_e)
    chunk_dep_layer = [0, 1, 2, 2]
    chunk_rows = E // 4
    aggs_hist = []
    for (pv, _pe) in layers:
        aggs = _sc_agg(v, e, src, dst, zero)
        aggs_hist.append(aggs)
        v = _layer_mlp(aggs, v, pv, bm=1000)
    edge_chunks = []
    for k in range(4):
        edge_chunks.append(
            _decode_chunk(e, dec_e_p, bm=2000, rows=chunk_rows,
                          row0=k * chunk_rows,
                          dep=aggs_hist[chunk_dep_layer[k]])[:, :dout_e])
    edge_out = jnp.concatenate(edge_chunks, axis=0)

    dec_v_p, dout_v = _pad_dec(dec_v)
    node_out = _mlp(v, dec_v_p, bm=1000)[:, :dout_v]
    return (node_out, edge_out)


# speedup vs baseline: 5.4437x; 1.1004x over previous
import jax
import jax.numpy as jnp
from jax import lax
from jax.experimental import pallas as pl
from jax.experimental.pallas import tpu as pltpu
from jax.experimental.pallas import tpu_sc as plsc

N = 10000
E = 320000
H = 128
EPS = 1e-7

NC = 2
NS = 16
IW = 64
EPC = E // NC
EPT = EPC // NS
FULL_STEPS = EPT // IW
TAIL = EPT - FULL_STEPS * IW
ROWS_PT = 624
ROWS_TAIL = N - NS * ROWS_PT


def _dot3(x, w):
    return jnp.dot(x.astype(jnp.bfloat16), w.astype(jnp.bfloat16),
                   preferred_element_type=jnp.float32)


def _sc_agg_body(v_hbm, e_hbm, vei_hbm, zero_hbm, out_hbm,
                 sidx, di0, di1, g0, g1, eb0, eb1, acc,
                 sg0, sg1, se0, se1, ss0, ss1, sd0, sd1):
    cid = lax.axis_index("c")
    sid = lax.axis_index("s")

    pltpu.sync_copy(zero_hbm.at[pl.ds(sid * ROWS_PT, ROWS_PT)],
                    acc.at[pl.ds(sid * ROWS_PT, ROWS_PT)])

    @pl.when(sid == 0)
    def _():
        pltpu.sync_copy(zero_hbm.at[pl.ds(NS * ROWS_PT, ROWS_TAIL)],
                        acc.at[pl.ds(NS * ROWS_PT, ROWS_TAIL)])

    base0 = cid * EPC + sid * EPT
    pltpu.sync_copy(vei_hbm.at[pl.ds(base0, EPT)], sidx)
    plsc.subcore_barrier()

    def gather_start(j, w, gb, sg):
        pltpu.async_copy(v_hbm.at[sidx.at[pl.ds(j * IW, w)]], gb, sg)

    def gather_wait(j, w, gb, sg):
        pltpu.make_async_copy(
            v_hbm.at[sidx.at[pl.ds(j * IW, w)]], gb, sg).wait()

    def e_start(j, w, ebuf, se):
        pltpu.async_copy(e_hbm.at[pl.ds(base0 + j * IW, w)], ebuf, se)

    def e_wait(j, w, ebuf, se):
        pltpu.make_async_copy(
            e_hbm.at[pl.ds(base0 + j * IW, w)], ebuf, se).wait()

    def d_start(j, w, dib, sd):
        pltpu.async_copy(vei_hbm.at[pl.ds(E + base0 + j * IW, w)], dib, sd)

    def d_wait(j, w, dib, sd):
        pltpu.make_async_copy(
            vei_hbm.at[pl.ds(E + base0 + j * IW, w)], dib, sd).wait()

    def scatter_start(gb, dib, ss):
        pltpu.async_copy(gb, acc.at[dib], ss, add=True)

    def scatter_wait(gb, dib, ss):
        pltpu.make_async_copy(gb, acc.at[dib], ss).wait()

    def compute(gb, ebuf, rows):
        @pl.loop(0, rows, step=2)
        def _row(r):
            for r2 in range(2):
                for c16 in range(H // 16):
                    slc = (pl.ds(r + r2, 1), pl.ds(c16 * 16, 16))
                    gb.at[slc][...] = (
                        jnp.maximum(gb.at[slc][...] + ebuf.at[slc][...], 0.0)
                        + EPS)

    def step(j, gb, ebuf, dib, sg, se, sd, ss, og, odi, oss):
        pj = j + 1

        @pl.when(pj < FULL_STEPS)
        def _():
            @pl.when(j >= 1)
            def _():
                scatter_wait(og, odi, oss)
            if ss is ss0:
                gather_start(pj, IW, g1, sg1)
                e_start(pj, IW, eb1, se1)
                d_start(pj, IW, di1, sd1)
            else:
                gather_start(pj, IW, g0, sg0)
                e_start(pj, IW, eb0, se0)
                d_start(pj, IW, di0, sd0)

        gather_wait(j, IW, gb, sg)
        e_wait(j, IW, ebuf, se)
        compute(gb, ebuf, IW)
        d_wait(j, IW, dib, sd)
        scatter_start(gb, dib, ss)

    gather_start(0, IW, g0, sg0)
    e_start(0, IW, eb0, se0)
    d_start(0, IW, di0, sd0)

    @pl.loop(0, FULL_STEPS // 2)
    def _pair(i):
        j0 = 2 * i
        step(j0, g0, eb0, di0, sg0, se0, sd0, ss0, g1, di1, ss1)
        step(j0 + 1, g1, eb1, di1, sg1, se1, sd1, ss1, g0, di0, ss0)

    scatter_wait(g0, di0, ss0)
    scatter_wait(g1, di1, ss1)

    gt = g0.at[pl.ds(0, TAIL)]
    et = eb0.at[pl.ds(0, TAIL)]
    dt = di0.at[pl.ds(0, TAIL)]
    pltpu.async_copy(v_hbm.at[sidx.at[pl.ds(FULL_STEPS * IW, TAIL)]],
                     gt, sg0).wait()
    pltpu.async_copy(e_hbm.at[pl.ds(base0 + FULL_STEPS * IW, TAIL)],
                     et, se0).wait()
    pltpu.async_copy(vei_hbm.at[pl.ds(E + base0 + FULL_STEPS * IW, TAIL)],
                     dt, sd0).wait()
    compute(g0, eb0, TAIL)
    pltpu.sync_copy(gt, acc.at[dt], add=True)

    plsc.subcore_barrier()
    pltpu.sync_copy(acc.at[pl.ds(sid * ROWS_PT, ROWS_PT)],
                    out_hbm.at[cid, pl.ds(sid * ROWS_PT, ROWS_PT)])

    @pl.when(sid == 0)
    def _():
        pltpu.sync_copy(acc.at[pl.ds(NS * ROWS_PT, ROWS_TAIL)],
                        out_hbm.at[cid, pl.ds(NS * ROWS_PT, ROWS_TAIL)])


def _sc_agg(v, e, vei, zero):
    mesh = plsc.VectorSubcoreMesh(core_axis_name="c", subcore_axis_name="s")
    k = pl.kernel(
        _sc_agg_body,
        mesh=mesh,
        out_type=jax.ShapeDtypeStruct((NC, N, H), jnp.float32),
        scratch_types=[
            pltpu.VMEM((EPT,), jnp.int32),
            pltpu.VMEM((IW,), jnp.int32),
            pltpu.VMEM((IW,), jnp.int32),
            pltpu.VMEM((IW, H), jnp.float32),
            pltpu.VMEM((IW, H), jnp.float32),
            pltpu.VMEM((IW, H), jnp.float32),
            pltpu.VMEM((IW, H), jnp.float32),
            pltpu.VMEM_SHARED((N, H), jnp.float32),
            pltpu.SemaphoreType.DMA,
            pltpu.SemaphoreType.DMA,
            pltpu.SemaphoreType.DMA,
            pltpu.SemaphoreType.DMA,
            pltpu.SemaphoreType.DMA,
            pltpu.SemaphoreType.DMA,
            pltpu.SemaphoreType.DMA,
            pltpu.SemaphoreType.DMA,
        ],
    )
    return k(v, e, vei, zero)


def _mlp_body(x_ref, w1_ref, b1_ref, w2_ref, b2_ref, o_ref):
    h = jnp.maximum(_dot3(x_ref[...], w1_ref[...]) + b1_ref[...], 0.0)
    o_ref[...] = _dot3(h, w2_ref[...]) + b2_ref[...]


def _mlp(x, p, bm):
    W1, b1, W2, b2 = p
    M, din = x.shape
    dh = W1.shape[1]
    dout = W2.shape[1]
    return pl.pallas_call(
        _mlp_body,
        grid=(M // bm,),
        in_specs=[
            pl.BlockSpec((bm, din), lambda i: (i, 0)),
            pl.BlockSpec((din, dh), lambda i: (0, 0)),
            pl.BlockSpec((1, dh), lambda i: (0, 0)),
            pl.BlockSpec((dh, dout), lambda i: (0, 0)),
            pl.BlockSpec((1, dout), lambda i: (0, 0)),
        ],
        out_specs=pl.BlockSpec((bm, dout), lambda i: (i, 0)),
        out_shape=jax.ShapeDtypeStruct((M, dout), jnp.float32),
    )(x, W1, b1.reshape(1, dh), W2, b2.reshape(1, dout))


def _layer_body(a_ref, v_ref, w1_ref, b1_ref, w2_ref, b2_ref, o_ref):
    x = a_ref[0] + a_ref[1]
    h = jnp.maximum(_dot3(x, w1_ref[...]) + b1_ref[...], 0.0)
    o_ref[...] = v_ref[...] + _dot3(h, w2_ref[...]) + b2_ref[...]


def _layer_mlp(aggs, v, p, bm):
    W1, b1, W2, b2 = p
    return pl.pallas_call(
        _layer_body,
        grid=(N // bm,),
        in_specs=[
            pl.BlockSpec((2, bm, H), lambda i: (0, i, 0)),
            pl.BlockSpec((bm, H), lambda i: (i, 0)),
            pl.BlockSpec((H, H), lambda i: (0, 0)),
            pl.BlockSpec((1, H), lambda i: (0, 0)),
            pl.BlockSpec((H, H), lambda i: (0, 0)),
            pl.BlockSpec((1, H), lambda i: (0, 0)),
        ],
        out_specs=pl.BlockSpec((bm, H), lambda i: (i, 0)),
        out_shape=jax.ShapeDtypeStruct((N, H), jnp.float32),
    )(aggs, v, W1, b1.reshape(1, H), W2, b2.reshape(1, H))


def _pad_dec(p):
    W1, b1, W2, b2 = p
    dh, dout = W2.shape
    dpad = 8
    W2p = jnp.zeros((dh, dpad), jnp.float32).at[:, :dout].set(W2)
    b2p = jnp.zeros((dpad,), jnp.float32).at[:dout].set(b2)
    return (W1, b1, W2p, b2p), dout


def _dec_dep_body(x_ref, w1_ref, b1_ref, w2_ref, b2_ref, dep_ref, o_ref):
    del dep_ref
    _mlp_body(x_ref, w1_ref, b1_ref, w2_ref, b2_ref, o_ref)


def _decode_chunk(x, p, bm, rows, row0, dep):
    W1, b1, W2, b2 = p
    dh = W1.shape[1]
    dout = W2.shape[1]
    blk0 = row0 // bm
    return pl.pallas_call(
        _dec_dep_body,
        grid=(rows // bm,),
        in_specs=[
            pl.BlockSpec((bm, H), lambda i: (i + blk0, 0)),
            pl.BlockSpec((H, dh), lambda i: (0, 0)),
            pl.BlockSpec((1, dh), lambda i: (0, 0)),
            pl.BlockSpec((dh, dout), lambda i: (0, 0)),
            pl.BlockSpec((1, dout), lambda i: (0, 0)),
            pl.BlockSpec((1, 8, H), lambda i: (0, 0, 0)),
        ],
        out_specs=pl.BlockSpec((bm, dout), lambda i: (i, 0)),
        out_shape=jax.ShapeDtypeStruct((rows, dout), jnp.float32),
    )(x, W1, b1.reshape(1, dh), W2, b2.reshape(1, dout), dep)


def kernel(v_x, v_edge_index, e_x, e_edge_index, enc_v, enc_e, layers,
           dec_v, dec_e):
    vei = v_edge_index.reshape(-1)
    zero = jnp.zeros((N, H), jnp.float32)

    v = _mlp(v_x, enc_v, bm=1000)
    e = _mlp(e_x, enc_e, bm=4000)

    dec_e_p, dout_e = _pad_dec(dec_e)
    chunk_dep_layer = [0, 1, 2, 2]
    chunk_rows = E // 4
    aggs_hist = []
    for (pv, _pe) in layers:
        aggs = _sc_agg(v, e, vei, zero)
        aggs_hist.append(aggs)
        v = _layer_mlp(aggs, v, pv, bm=1000)
    edge_chunks = []
    for k in range(4):
        edge_chunks.append(
            _decode_chunk(e, dec_e_p, bm=4000, rows=chunk_rows,
                          row0=k * chunk_rows,
                          dep=aggs_hist[chunk_dep_layer[k]])[:, :dout_e])
    edge_out = jnp.concatenate(edge_chunks, axis=0)

    dec_v_p, dout_v = _pad_dec(dec_v)
    node_out = _mlp(v, dec_v_p, bm=1000)[:, :dout_v]
    return (node_out, edge_out)
